# baseline (speedup 1.0000x reference)
import functools

import jax
import jax.numpy as jnp
from jax import lax
from jax.experimental import pallas as pl
from jax.experimental.pallas import tpu as pltpu
from jax.experimental.pallas import tpu_sc as plsc

MEM = 1024
D = 64
N = 16384
R = 1024
NB = N // R
SCALE = 1.25
NEG_INF = float("-inf")

NW = 32
RW = N // NW
SLOTS_PER_TILE = MEM // 16


def _pass_a(q_ref, keys_ref, sm_ref, uq_ref, qf_ref, m_ref, g_ref,
            cmax_ref, csum_ref, misc_ref):
    i = pl.program_id(0)
    q = q_ref[0]
    qt = q.reshape(D, R).T
    ss = jnp.sum(qt * qt, axis=1, keepdims=True)
    qf = qt / jnp.maximum(jnp.sqrt(ss), 1e-12)
    qf_ref[...] = qf
    keys = keys_ref[...]

    l = lax.dot_general(qf, keys, (((1,), (1,)), ((), ())),
                        preferred_element_type=jnp.float32) * SCALE
    m = jnp.max(l, axis=1)
    expl = jnp.exp(l - m[:, None])
    rs = jnp.sum(expl, axis=1)
    sm = expl * (1.0 / rs)[:, None]
    sm_ref[...] = sm

    cols = lax.broadcasted_iota(jnp.int32, (R, MEM), 1)
    gi = jnp.min(jnp.where(l == m[:, None], cols, MEM), axis=1)
    mask1 = cols == gi[:, None]
    l2 = jnp.where(mask1, NEG_INF, l)
    m2 = jnp.max(l2, axis=1)
    g2 = jnp.min(jnp.where(l2 == m2[:, None], cols, MEM), axis=1)
    mask2 = cols == g2[:, None]

    m_ref[...] = m[None, None, :]
    g_ref[...] = gi[None, None, :]

    cm = lax.dot_general(sm, keys, (((1,), (0,)), ((), ())),
                         preferred_element_type=jnp.float32)
    uq = jnp.concatenate([qf, cm], axis=1)
    uq_ref[...] = uq.T.reshape(1, 2 * D, 32, 32)

    ksq = jnp.sum(keys * keys, axis=1, keepdims=True)
    ksum = jnp.sum(keys, axis=1, keepdims=True)
    slotstats = jnp.concatenate([ksq, ksum], axis=1)
    st1 = lax.dot_general(mask1.astype(jnp.float32), slotstats,
                          (((1,), (0,)), ((), ())),
                          preferred_element_type=jnp.float32)
    st2 = lax.dot_general(mask2.astype(jnp.float32), slotstats,
                          (((1,), (0,)), ((), ())),
                          preferred_element_type=jnp.float32)
    qsq = jnp.sum(qf * qf, axis=1)
    qrow = jnp.sum(qf, axis=1)
    dot1 = 2.0 * 0.8 * m
    dot2 = 2.0 * 0.8 * m2
    dsq1 = qsq - dot1 + st1[:, 0]
    dsq2 = qsq - dot2 + st2[:, 0]
    comp_p = jnp.sum(dsq1)
    eps2 = 2e-6
    epsq = 64e-12
    dp = jnp.sqrt(dsq1 + eps2 * (qrow - st1[:, 1]) + epsq)
    dn = jnp.sqrt(dsq2 + eps2 * (qrow - st2[:, 1]) + epsq)
    sep_p = jnp.sum(jnp.maximum(dp - dn + 1.0, 0.0))

    bmax = jnp.max(l, axis=0)[None, :]
    K = jnp.max(m)
    w = jnp.exp(m - K)
    bsum = lax.dot_general(w[None, :], expl, (((1,), (0,)), ((), ())),
                           preferred_element_type=jnp.float32)

    @pl.when(i == 0)
    def _():
        cmax_ref[...] = jnp.full((1, MEM), NEG_INF, jnp.float32)
        csum_ref[...] = jnp.zeros((1, MEM), jnp.float32)
        misc_ref[...] = jnp.zeros((1, 128), jnp.float32)

    old_m = cmax_ref[...]
    old_s = csum_ref[...]
    new_m = jnp.maximum(old_m, bmax)
    new_s = old_s * jnp.exp(old_m - new_m) + bsum * jnp.exp(K - new_m)
    cmax_ref[...] = new_m
    csum_ref[...] = new_s

    lanes = lax.broadcasted_iota(jnp.int32, (1, 128), 1)
    contrib = (jnp.where(lanes == 0, comp_p, 0.0)
               + jnp.where(lanes == 1, sep_p, 0.0))
    misc_ref[...] = misc_ref[...] + contrib

    @pl.when(i == NB - 1)
    def _():
        acc = misc_ref[...]
        scale_vec = jnp.where(lanes == 0, 1.0 / (N * D),
                              jnp.where(lanes == 1, 1.0 / N, 0.0))
        gmax = jnp.max(new_m)
        misc_ref[...] = acc * scale_vec + jnp.where(lanes == 2, gmax, 0.0)


HALF = MEM // 2
QCH = 128


def _sc_update_body(qf_hbm, m_hbm, g_hbm, gv_hbm, acc_hbm,
                    qf_v, m_v, g_v, gv_v, acc_v):
    c = lax.axis_index("c")
    s = lax.axis_index("s")
    wid = s * 2 + c
    base = wid * RW
    pltpu.sync_copy(m_hbm.at[pl.ds(base, RW)], m_v)
    pltpu.sync_copy(g_hbm.at[pl.ds(base, RW)], g_v)
    pltpu.sync_copy(gv_hbm, gv_v)

    gmax = gv_v[...]

    def zbody(j, carry):
        for k in range(D // 16):
            acc_v[j, pl.ds(k * 16, 16)] = jnp.zeros((16,), jnp.float32)
        return carry

    for p in range(2):
        lo = p * HALF
        lax.fori_loop(0, HALF, zbody, 0)
        for cch in range(RW // QCH):
            pltpu.sync_copy(qf_hbm.at[pl.ds(base + cch * QCH, QCH)], qf_v)

            def gbody(gi, carry, _cch=cch, _lo=lo):
                off = _cch * QCH + gi * 16
                wvec = jnp.exp(m_v[pl.ds(off, 16)] - gmax)
                gvec = g_v[pl.ds(off, 16)]
                for j in range(16):
                    r = gi * 16 + j
                    wgt = wvec[j]
                    slot = gvec[j]
                    slotl = slot - _lo

                    @pl.when((slot >= _lo) & (slot < _lo + HALF))
                    def _():
                        for k in range(D // 16):
                            sl = pl.ds(k * 16, 16)
                            acc_v[slotl, sl] = (acc_v[slotl, sl]
                                                + qf_v[r, sl] * wgt)
                return carry

            lax.fori_loop(0, QCH // 16, gbody, 0)
        pltpu.sync_copy(acc_v, acc_hbm.at[wid, pl.ds(lo, HALF)])


_sc_update = functools.partial(
    pl.kernel,
    mesh=plsc.VectorSubcoreMesh(core_axis_name="c", subcore_axis_name="s",
                                num_cores=2, num_subcores=16),
    out_type=jax.ShapeDtypeStruct((NW, MEM, D), jnp.float32),
    scratch_types=[
        pltpu.VMEM((QCH, D), jnp.float32),
        pltpu.VMEM((RW,), jnp.float32),
        pltpu.VMEM((RW,), jnp.int32),
        pltpu.VMEM((16,), jnp.float32),
        pltpu.VMEM((HALF, D), jnp.float32),
    ],
)(_sc_update_body)


def _pass_b(qf_ref, keys_ref, cmax_ref, csum_ref, cmax_t_ref, csum_t_ref,
            acc_ref, sq_ref, um_ref):
    i = pl.program_id(0)
    qf = qf_ref[...]
    keys = keys_ref[...]
    l = lax.dot_general(qf, keys, (((1,), (1,)), ((), ())),
                        preferred_element_type=jnp.float32) * SCALE
    c_row = cmax_ref[...] + jnp.log(csum_ref[...])
    sq = jnp.exp(l - c_row)
    sq_ref[...] = sq

    @pl.when(i == NB - 1)
    def _():
        gmax = jnp.max(cmax_ref[...])
        scale = jnp.exp(gmax - cmax_t_ref[...]) / csum_t_ref[...]
        qu = jnp.sum(acc_ref[...], axis=0) * scale
        um = 0.5 * keys + 0.5 * qu
        nrm = jnp.sqrt(jnp.sum(um * um, axis=1, keepdims=True))
        um_ref[...] = um / jnp.maximum(nrm, 1e-12)


def kernel(query, keys):
    b, dims, h, w = query.shape

    sm, uq, qf, m3, g3, cmax, csum, misc = pl.pallas_call(
        _pass_a,
        grid=(NB,),
        in_specs=[
            pl.BlockSpec((1, D, 32, 32), lambda i: (i, 0, 0, 0)),
            pl.BlockSpec((MEM, D), lambda i: (0, 0)),
        ],
        out_specs=[
            pl.BlockSpec((R, MEM), lambda i: (i, 0)),
            pl.BlockSpec((1, 2 * D, 32, 32), lambda i: (i, 0, 0, 0)),
            pl.BlockSpec((R, D), lambda i: (i, 0)),
            pl.BlockSpec((1, 1, R), lambda i: (i, 0, 0)),
            pl.BlockSpec((1, 1, R), lambda i: (i, 0, 0)),
            pl.BlockSpec((1, MEM), lambda i: (0, 0)),
            pl.BlockSpec((1, MEM), lambda i: (0, 0)),
            pl.BlockSpec((1, 128), lambda i: (0, 0)),
        ],
        out_shape=[
            jax.ShapeDtypeStruct((N, MEM), jnp.float32),
            jax.ShapeDtypeStruct((b, 2 * D, h, w), jnp.float32),
            jax.ShapeDtypeStruct((N, D), jnp.float32),
            jax.ShapeDtypeStruct((NB, 1, R), jnp.float32),
            jax.ShapeDtypeStruct((NB, 1, R), jnp.int32),
            jax.ShapeDtypeStruct((1, MEM), jnp.float32),
            jax.ShapeDtypeStruct((1, MEM), jnp.float32),
            jax.ShapeDtypeStruct((1, 128), jnp.float32),
        ],
    )(query, keys)

    m_flat = m3.reshape(N)
    g_flat = g3.reshape(N)
    gv = jnp.broadcast_to(misc[0:1, 2], (16,))

    acc = _sc_update(qf, m_flat, g_flat, gv)

    sq, um = pl.pallas_call(
        _pass_b,
        grid=(NB,),
        in_specs=[
            pl.BlockSpec((R, D), lambda i: (i, 0)),
            pl.BlockSpec((MEM, D), lambda i: (0, 0)),
            pl.BlockSpec((1, MEM), lambda i: (0, 0)),
            pl.BlockSpec((1, MEM), lambda i: (0, 0)),
            pl.BlockSpec((MEM, 1), lambda i: (0, 0)),
            pl.BlockSpec((MEM, 1), lambda i: (0, 0)),
            pl.BlockSpec((NW, MEM, D), lambda i: (0, 0, 0)),
        ],
        out_specs=[
            pl.BlockSpec((R, MEM), lambda i: (i, 0)),
            pl.BlockSpec((MEM, D), lambda i: (0, 0)),
        ],
        out_shape=[
            jax.ShapeDtypeStruct((N, MEM), jnp.float32),
            jax.ShapeDtypeStruct((MEM, D), jnp.float32),
        ],
    )(qf, keys, cmax, csum,
      jnp.reshape(cmax, (MEM, 1)), jnp.reshape(csum, (MEM, 1)), acc)

    comp = misc[0, 0]
    sep = misc[0, 1]
    return (uq, um, sq, sm, sep, comp)
